# Initial kernel scaffold; baseline (speedup 1.0000x reference)
import jax
import jax.numpy as jnp
from jax.experimental import pallas as pl


def kernel(x):
    raise NotImplementedError("write your pallas kernel here")


# speedup vs baseline: 2.8802x; 2.8802x over previous
import functools

import jax
import jax.numpy as jnp
from jax import lax
from jax.experimental import pallas as pl
from jax.experimental.pallas import tpu as pltpu
from jax.experimental.pallas import tpu_sc as plsc

ROWS = 8192
COLS = 2048
REP = 4
NC = 2
NS = 16
NW = NC * NS
ROWS_PER_W = ROWS // NW
CH = 32
NCHUNK = ROWS_PER_W // CH


def _repeat_kernel(x_hbm, out_hbm, buf, idx0, idx1, idx2, idx3, sem):
    wid = lax.axis_index("s") * NC + lax.axis_index("c")
    base0 = wid * ROWS_PER_W
    idx_refs = (idx0, idx1, idx2, idx3)

    def chunk_body(g, carry):
        base = base0 + g * CH
        pltpu.sync_copy(x_hbm.at[pl.ds(base, CH)], buf)
        for t in range(CH // 16):
            rows = base + t * 16 + lax.iota(jnp.int32, 16)
            for j in range(REP):
                idx_refs[j][pl.ds(t * 16, 16)] = rows * REP + j
        copies = [
            pltpu.async_copy(buf, out_hbm.at[idx_refs[j]], sem)
            for j in range(REP)
        ]
        for c in copies:
            c.wait()
        return carry

    lax.fori_loop(0, NCHUNK, chunk_body, 0)


@jax.jit
def _repeat_interleave(x):
    mesh = plsc.VectorSubcoreMesh(core_axis_name="c", subcore_axis_name="s")
    k = functools.partial(
        pl.kernel,
        out_type=jax.ShapeDtypeStruct((ROWS * REP, COLS), jnp.float32),
        mesh=mesh,
        scratch_types=[
            pltpu.VMEM((CH, COLS), jnp.float32),
            pltpu.VMEM((CH,), jnp.int32),
            pltpu.VMEM((CH,), jnp.int32),
            pltpu.VMEM((CH,), jnp.int32),
            pltpu.VMEM((CH,), jnp.int32),
            pltpu.SemaphoreType.DMA,
        ],
    )(_repeat_kernel)
    return k(x)


def kernel(x):
    return _repeat_interleave(x)
